# baseline (speedup 1.0000x reference)
import dataclasses
import functools
import math

import jax
import jax.numpy as jnp
import numpy as np
from jax import lax
from jax.experimental import pallas as pl
from jax.experimental.pallas import tpu as pltpu
from jax.experimental.pallas import tpu_sc as plsc

N = 10000
E = 320000
EP = 327680
NUM_BASIS = 10
MAX_RADIUS = 5.0

_MESH = plsc.VectorSubcoreMesh(core_axis_name="c", subcore_axis_name="s")
_SC_PARAMS = pltpu.CompilerParams()
if "needs_layout_passes" in pltpu.CompilerParams.__dataclass_fields__:
    _SC_PARAMS = dataclasses.replace(_SC_PARAMS, needs_layout_passes=False)
_NS = 16
_EPT_G = E // 32
_ROWS_PT = 632
_NP = _ROWS_PT * _NS
_CHUNK = 80

_BE = 2048
_BN = 2000


def _silu(v):
    return v * jax.nn.sigmoid(v)


def _geom(px, py, pz, src, dst):
    @functools.partial(
        pl.kernel,
        out_type=jax.ShapeDtypeStruct((EP,), jnp.float32),
        mesh=_MESH,
        compiler_params=_SC_PARAMS,
        scratch_types=[
            pltpu.VMEM((N,), jnp.float32),
            pltpu.VMEM((N,), jnp.float32),
            pltpu.VMEM((N,), jnp.float32),
            pltpu.VMEM((_EPT_G,), jnp.int32),
            pltpu.VMEM((_EPT_G,), jnp.int32),
            pltpu.VMEM((_EPT_G,), jnp.float32),
        ],
    )
    def k(px_h, py_h, pz_h, src_h, dst_h, out_h, px_v, py_v, pz_v, si_v, di_v, o_v):
        c = lax.axis_index("c")
        s = lax.axis_index("s")
        base = (c * _NS + s) * _EPT_G
        pltpu.sync_copy(px_h, px_v)
        pltpu.sync_copy(py_h, py_v)
        pltpu.sync_copy(pz_h, pz_v)
        pltpu.sync_copy(src_h.at[pl.ds(base, _EPT_G)], si_v)
        pltpu.sync_copy(dst_h.at[pl.ds(base, _EPT_G)], di_v)

        def _do16(i):
            sl = pl.ds(i, 16)
            a = si_v[sl]
            b = di_v[sl]
            dx = plsc.load_gather(px_v, [a]) - plsc.load_gather(px_v, [b])
            dy = plsc.load_gather(py_v, [a]) - plsc.load_gather(py_v, [b])
            dz = plsc.load_gather(pz_v, [a]) - plsc.load_gather(pz_v, [b])
            o_v[sl] = dx * dx + dy * dy + dz * dz

        @pl.loop(0, _EPT_G - 16, step=32)
        def _(i):
            _do16(i)
            _do16(i + 16)

        _do16(_EPT_G - 16)
        pltpu.sync_copy(o_v, out_h.at[pl.ds(base, _EPT_G)])

    return k(px, py, pz, src, dst)


_NR = EP // 128


def _sparse_layer(split, xl, we, src2, dst2):
    nchunks = (_NR * 2) // _NS if split else _NR // _NS
    nblocks = nchunks // 32

    @functools.partial(
        pl.kernel,
        out_type=jax.ShapeDtypeStruct((2 * _NP, 128), jnp.float32),
        mesh=_MESH,
        compiler_params=_SC_PARAMS,
        scratch_types=[
            pltpu.VMEM((32, 64), jnp.int32),
            pltpu.VMEM((32, 64), jnp.int32),
            pltpu.VMEM((64, 128), jnp.float32),
            pltpu.VMEM((64, 128), jnp.float32),
            pltpu.VMEM((64, 64), jnp.int32),
            pltpu.VMEM((64, 64), jnp.int32),
            pltpu.SemaphoreType.DMA,
            pltpu.SemaphoreType.DMA,
            pltpu.SemaphoreType.DMA,
            pltpu.SemaphoreType.DMA,
            pltpu.VMEM_SHARED((_NP, 128), jnp.float32),
        ],
    )
    def k(xl_h, we_h, src_h, dst_h, out_h, si_b, di_b, r0, r1, e0, e1,
          g0, g1, w0, w1, agg_sh):
        c = lax.axis_index("c")
        s = lax.axis_index("s")
        cbase = (s * nchunks) if split else ((c * _NS + s) * nchunks)
        ebase = cbase * 64

        @pl.loop(0, 64)
        def _(i):
            for jj in range(8):
                r0[i, pl.ds(jj * 16, 16)] = jnp.zeros((16,), jnp.float32)

        zbase = s * _ROWS_PT
        for t in range(9):
            pltpu.sync_copy(r0, agg_sh.at[pl.ds(zbase + t * 64, 64)])
        pltpu.sync_copy(r0.at[pl.ds(0, 56)], agg_sh.at[pl.ds(zbase + 576, 56)])
        plsc.subcore_barrier()

        web = (c * EP if split else 0) + ebase
        off = c * N

        @pl.loop(0, nblocks)
        def _(b):
            brow = cbase + b * 32
            pltpu.sync_copy(src_h.at[pl.ds(brow, 32)], si_b)
            pltpu.sync_copy(dst_h.at[pl.ds(brow, 32)], di_b)
            if split:

                @pl.loop(0, 32)
                def _(i):
                    for jj in range(4):
                        sl = pl.ds(jj * 16, 16)
                        si_b[i, sl] = si_b[i, sl] + off

            eb = web + b * 2048

            def gcopy(buf, sem, cc):
                return pltpu.make_async_copy(xl_h.at[si_b.at[cc]], buf, sem)

            def wcopy(buf, sem, cc):
                return pltpu.make_async_copy(
                    we_h.at[pl.ds(eb + cc * 64, 64)], buf, sem
                )

            gcopy(r0, g0, 0).start()
            wcopy(e0, w0, 0).start()
            gcopy(r1, g1, 1).start()
            wcopy(e1, w1, 1).start()

            def slot(rv, ev, gs, ws, tt, nxt):
                gcopy(rv, gs, tt).wait()
                wcopy(ev, ws, tt).wait()

                mask = jnp.int32(-65536)

                @pl.loop(0, 64, step=2)
                def _(i):
                    for ii in range(2):
                        for g in range(4):
                            wi = ev[i + ii, pl.ds(g * 16, 16)]
                            wa = plsc.bitcast(
                                lax.shift_left(wi, 16), jnp.float32
                            )
                            wb = plsc.bitcast(wi & mask, jnp.float32)
                            sa = pl.ds(g * 32, 16)
                            sb = pl.ds(g * 32 + 16, 16)
                            rv[i + ii, sa] = rv[i + ii, sa] * wa
                            rv[i + ii, sb] = rv[i + ii, sb] * wb

                pltpu.sync_copy(rv, agg_sh.at[di_b.at[tt]], add=True)

                @pl.when(nxt < 32)
                def _():
                    gcopy(rv, gs, nxt).start()
                    wcopy(ev, ws, nxt).start()

            @pl.loop(0, 32, step=2)
            def _(t):
                slot(r0, e0, g0, w0, t, t + 2)
                slot(r1, e1, g1, w1, t + 1, t + 3)

        plsc.subcore_barrier()
        pltpu.sync_copy(
            agg_sh.at[pl.ds(zbase, _ROWS_PT)],
            out_h.at[pl.ds(c * _NP + zbase, _ROWS_PT)],
        )

    return k(xl, we, src2, dst2)


def _radial(dl2, layers):
    d2 = dl2.reshape(EP, 1)
    vals = np.linspace(0.0, MAX_RADIUS, NUM_BASIS).astype(np.float32)
    step = float(vals[1] - vals[0])
    hs = [w["fc_w2h"].shape[0] for w in layers]

    def body(d2_ref, *refs):
        wrefs = refs[:9]
        orefs = refs[9:]
        el = jnp.sqrt(d2_ref[...] + 1e-9)
        vgrid = (
            lax.broadcasted_iota(jnp.int32, (1, NUM_BASIS), 1).astype(jnp.float32)
            * step
        )
        diff = (el - vgrid) / step
        emb = (jnp.exp(-diff * diff) / 1.12) * (NUM_BASIS ** 0.5)
        u = 2.0 * (el / MAX_RADIUS - 1.0)
        y = (1.0 - jnp.cos(np.float32(math.pi) * u)) / 2.0
        y = jnp.where(u > 0.0, 0.0, y)
        y = jnp.where(u < -1.0, 1.0, y)
        for li in range(3):
            f0, f1, f2 = wrefs[3 * li : 3 * li + 3]
            h0 = _silu(jnp.dot(emb, f0[...], preferred_element_type=jnp.float32) / np.sqrt(NUM_BASIS))
            h1 = _silu(jnp.dot(h0, f1[...], preferred_element_type=jnp.float32) / np.sqrt(128.0))
            for hh in range(hs[li]):
                w = jnp.dot(h1, f2[hh], preferred_element_type=jnp.float32) / np.sqrt(128.0)
                wy = w * y
                words = []
                for g in range(4):
                    a = wy[:, 32 * g : 32 * g + 16]
                    b = wy[:, 32 * g + 16 : 32 * g + 32]
                    ai = lax.shift_right_logical(
                        lax.bitcast_convert_type(
                            a.astype(jnp.bfloat16).astype(jnp.float32),
                            jnp.int32,
                        ),
                        16,
                    )
                    bi = lax.bitcast_convert_type(
                        b.astype(jnp.bfloat16).astype(jnp.float32), jnp.int32
                    ) & jnp.int32(-65536)
                    words.append(ai | bi)
                packed = jnp.concatenate(words, axis=1)
                if hs[li] == 1:
                    orefs[li][...] = packed
                else:
                    orefs[li][hh] = packed

    in_specs = [pl.BlockSpec((_BE, 1), lambda i: (i, 0))]
    args = [d2]
    for w in layers:
        args += [w["fc_w0"], w["fc_w1"], w["fc_w2h"]]
        in_specs += [
            pl.BlockSpec((NUM_BASIS, 128), lambda i: (0, 0)),
            pl.BlockSpec((128, 128), lambda i: (0, 0)),
            pl.BlockSpec((w["fc_w2h"].shape[0], 128, 128), lambda i: (0, 0, 0)),
        ]
    out_shapes = []
    out_specs = []
    for h in hs:
        if h == 1:
            out_shapes.append(jax.ShapeDtypeStruct((EP, 64), jnp.int32))
            out_specs.append(pl.BlockSpec((_BE, 64), lambda i: (i, 0)))
        else:
            out_shapes.append(jax.ShapeDtypeStruct((h, EP, 64), jnp.int32))
            out_specs.append(pl.BlockSpec((h, _BE, 64), lambda i: (0, i, 0)))
    outs = pl.pallas_call(
        body,
        grid=(EP // _BE,),
        in_specs=in_specs,
        out_specs=out_specs,
        out_shape=out_shapes,
    )(*args)
    return [o.reshape(-1, 64) for o in outs]


def _pre(h, wsc, w1h, h2=None, p=None):
    d_in = h.shape[1]
    d_out = wsc.shape[1]
    H = w1h.shape[0]

    def body(*refs):
        if p is None:
            h_ref, wsc_ref, w1_ref, xs_ref, xl_ref = refs
            hb = h_ref[...]
        else:
            h_ref, h2_ref, p_ref, wsc_ref, w1_ref, xs_ref, xl_ref = refs
            pv = p_ref[0, 0]
            hb = pv * h_ref[...] + (1.0 - pv) * h2_ref[...]
        c_s = np.float32(math.sin(math.pi / 8))
        xs_ref[...] = c_s * (
            jnp.dot(hb, wsc_ref[...], preferred_element_type=jnp.float32) / np.sqrt(d_in)
        )
        for k in range(H):
            xl_ref[k] = (
                jnp.dot(hb, w1_ref[k], preferred_element_type=jnp.float32) / np.sqrt(d_in)
            )

    in_specs = [pl.BlockSpec((_BN, d_in), lambda i: (i, 0))]
    args = [h]
    if p is not None:
        in_specs += [
            pl.BlockSpec((_BN, d_in), lambda i: (i, 0)),
            pl.BlockSpec((1, 1), lambda i: (0, 0)),
        ]
        args += [h2, p.reshape(1, 1)]
    in_specs += [
        pl.BlockSpec((d_in, d_out), lambda i: (0, 0)),
        pl.BlockSpec((H, d_in, 128), lambda i: (0, 0, 0)),
    ]
    args += [wsc, w1h]
    xs, xl = pl.pallas_call(
        body,
        grid=(N // _BN,),
        in_specs=in_specs,
        out_specs=[
            pl.BlockSpec((_BN, d_out), lambda i: (i, 0)),
            pl.BlockSpec((H, _BN, 128), lambda i: (0, i, 0)),
        ],
        out_shape=[
            jax.ShapeDtypeStruct((N, d_out), jnp.float32),
            jax.ShapeDtypeStruct((H, N, 128), jnp.float32),
        ],
    )(*args)
    return xs, xl.reshape(H * N, 128)


def _combine(agg, xs, w2h, d_in, act):
    d_out = xs.shape[1]
    H = w2h.shape[0]
    a = agg.reshape(2, _NP, 128)

    def body(a_ref, xs_ref, w2_ref, o_ref):
        c_x = np.float32(math.cos(math.pi / 8))
        if H == 1:
            ag = (a_ref[0] + a_ref[1]) / np.sqrt(32.0)
            out = jnp.dot(ag, w2_ref[0], preferred_element_type=jnp.float32)
        else:
            out = jnp.dot(
                a_ref[0] / np.sqrt(32.0), w2_ref[0], preferred_element_type=jnp.float32
            ) + jnp.dot(
                a_ref[1] / np.sqrt(32.0), w2_ref[1], preferred_element_type=jnp.float32
            )
        o = xs_ref[...] + c_x * (out / np.sqrt(d_in))
        if act:
            o = _silu(o)
        o_ref[...] = o

    return pl.pallas_call(
        body,
        grid=(N // _BN,),
        in_specs=[
            pl.BlockSpec((2, _BN, 128), lambda i: (0, i, 0)),
            pl.BlockSpec((_BN, d_out), lambda i: (i, 0)),
            pl.BlockSpec((H, 128, d_out), lambda i: (0, 0, 0)),
        ],
        out_specs=pl.BlockSpec((_BN, d_out), lambda i: (i, 0)),
        out_shape=jax.ShapeDtypeStruct((N, d_out), jnp.float32),
    )(a, xs, w2h)


def _prep_params(layers):
    out = []
    for lp in layers:
        d_in = lp["W_sc"].shape[0]
        H = d_in // 128
        w1 = lp["W_lin1"].reshape(d_in, H, 128)
        f2 = lp["fc_w2"].reshape(128, H, 128)
        out.append(
            {
                "W_sc": lp["W_sc"],
                "W_lin1h": jnp.transpose(w1, (1, 0, 2)),
                "fc_w0": lp["fc_w0"],
                "fc_w1": lp["fc_w1"],
                "fc_w2h": jnp.transpose(f2, (1, 0, 2)),
                "W_lin2h": lp["W_lin2"].reshape(H, 128, -1),
                "H": H,
                "d_in": d_in,
            }
        )
    return out


def _network(h, pos, srcp, src2, dst2, layers, h2=None, p=None):
    dl2 = _geom(pos[:, 0], pos[:, 1], pos[:, 2], srcp[0], srcp[1])
    wes = _radial(dl2, layers)
    for li, lp in enumerate(layers):
        if li == 0 and p is not None:
            xs, xl = _pre(h, lp["W_sc"], lp["W_lin1h"], h2=h2, p=p)
        else:
            xs, xl = _pre(h, lp["W_sc"], lp["W_lin1h"])
        agg = _sparse_layer(lp["H"] == 2, xl, wes[li], src2, dst2)
        h = _combine(agg, xs, lp["W_lin2h"], lp["d_in"], act=(li < 2))
    return h


def kernel(x, x_final_state, pos, pos_final_state, pos_interpolated_transition_state, p, edge_index, batch, params):
    src = edge_index[0].astype(jnp.int32)
    dst = edge_index[1].astype(jnp.int32)
    srcp = jnp.concatenate([src, jnp.zeros((EP - E,), jnp.int32)])
    dstp = jnp.concatenate([dst, jnp.full((EP - E,), N, jnp.int32)])
    src2, dst2 = srcp.reshape(EP // 64, 64), dstp.reshape(EP // 64, 64)
    sp = (srcp, dstp)

    net_i = _prep_params(params["net_init"])
    net_f = _prep_params(params["net_final"])
    net_ts = _prep_params(params["net_ts"])
    out_i = _network(x, pos, sp, src2, dst2, net_i)
    out_f = _network(x_final_state, pos_final_state, sp, src2, dst2, net_f)
    out_ts = _network(
        out_i,
        pos_interpolated_transition_state,
        sp,
        src2,
        dst2,
        net_ts,
        h2=out_f,
        p=p[0],
    )
    return out_ts


# speedup vs baseline: 1.1749x; 1.1749x over previous
import dataclasses
import functools
import math

import jax
import jax.numpy as jnp
import numpy as np
from jax import lax
from jax.experimental import pallas as pl
from jax.experimental.pallas import tpu as pltpu
from jax.experimental.pallas import tpu_sc as plsc

N = 10000
E = 320000
EP = 327680
NUM_BASIS = 10
MAX_RADIUS = 5.0

_MESH = plsc.VectorSubcoreMesh(core_axis_name="c", subcore_axis_name="s")
_SC_PARAMS = pltpu.CompilerParams()
if "needs_layout_passes" in pltpu.CompilerParams.__dataclass_fields__:
    _SC_PARAMS = dataclasses.replace(_SC_PARAMS, needs_layout_passes=False)
_NS = 16
_EPT_G = E // 32
_ROWS_PT = 632
_NP = _ROWS_PT * _NS
_CHUNK = 80

_BE = 2048
_BN = 2000


def _silu(v):
    return v * jax.nn.sigmoid(v)


def _geom(px, py, pz, src, dst):
    @functools.partial(
        pl.kernel,
        out_type=jax.ShapeDtypeStruct((EP,), jnp.float32),
        mesh=_MESH,
        compiler_params=_SC_PARAMS,
        scratch_types=[
            pltpu.VMEM((N,), jnp.float32),
            pltpu.VMEM((N,), jnp.float32),
            pltpu.VMEM((N,), jnp.float32),
            pltpu.VMEM((_EPT_G,), jnp.int32),
            pltpu.VMEM((_EPT_G,), jnp.int32),
            pltpu.VMEM((_EPT_G,), jnp.float32),
        ],
    )
    def k(px_h, py_h, pz_h, src_h, dst_h, out_h, px_v, py_v, pz_v, si_v, di_v, o_v):
        c = lax.axis_index("c")
        s = lax.axis_index("s")
        base = (c * _NS + s) * _EPT_G
        pltpu.sync_copy(px_h, px_v)
        pltpu.sync_copy(py_h, py_v)
        pltpu.sync_copy(pz_h, pz_v)
        pltpu.sync_copy(src_h.at[pl.ds(base, _EPT_G)], si_v)
        pltpu.sync_copy(dst_h.at[pl.ds(base, _EPT_G)], di_v)

        def _do16(i):
            sl = pl.ds(i, 16)
            a = si_v[sl]
            b = di_v[sl]
            dx = plsc.load_gather(px_v, [a]) - plsc.load_gather(px_v, [b])
            dy = plsc.load_gather(py_v, [a]) - plsc.load_gather(py_v, [b])
            dz = plsc.load_gather(pz_v, [a]) - plsc.load_gather(pz_v, [b])
            o_v[sl] = dx * dx + dy * dy + dz * dz

        @pl.loop(0, _EPT_G - 16, step=32)
        def _(i):
            _do16(i)
            _do16(i + 16)

        _do16(_EPT_G - 16)
        pltpu.sync_copy(o_v, out_h.at[pl.ds(base, _EPT_G)])

    return k(px, py, pz, src, dst)


_NR = EP // 128


def _sparse_layer(split, xl, we, src2, dst2):
    nchunks = (_NR * 2) // _NS if split else _NR // _NS
    nblocks = nchunks // 32

    @functools.partial(
        pl.kernel,
        out_type=jax.ShapeDtypeStruct((2 * _NP, 128), jnp.float32),
        mesh=_MESH,
        compiler_params=_SC_PARAMS,
        scratch_types=[
            pltpu.VMEM((32, 64), jnp.int32),
            pltpu.VMEM((32, 64), jnp.int32),
            pltpu.VMEM((64, 128), jnp.float32),
            pltpu.VMEM((64, 128), jnp.float32),
            pltpu.VMEM((64, 128), jnp.float32),
            pltpu.VMEM((64, 128), jnp.float32),
            pltpu.SemaphoreType.DMA,
            pltpu.SemaphoreType.DMA,
            pltpu.SemaphoreType.DMA,
            pltpu.SemaphoreType.DMA,
            pltpu.VMEM_SHARED((_NP, 128), jnp.float32),
        ],
    )
    def k(xl_h, we_h, src_h, dst_h, out_h, si_b, di_b, r0, r1, e0, e1,
          g0, g1, w0, w1, agg_sh):
        c = lax.axis_index("c")
        s = lax.axis_index("s")
        cbase = (s * nchunks) if split else ((c * _NS + s) * nchunks)
        ebase = cbase * 64

        @pl.loop(0, 64)
        def _(i):
            for jj in range(8):
                r0[i, pl.ds(jj * 16, 16)] = jnp.zeros((16,), jnp.float32)

        zbase = s * _ROWS_PT
        for t in range(9):
            pltpu.sync_copy(r0, agg_sh.at[pl.ds(zbase + t * 64, 64)])
        pltpu.sync_copy(r0.at[pl.ds(0, 56)], agg_sh.at[pl.ds(zbase + 576, 56)])
        plsc.subcore_barrier()

        web = (c * EP if split else 0) + ebase
        off = c * N

        @pl.loop(0, nblocks)
        def _(b):
            brow = cbase + b * 32
            pltpu.sync_copy(src_h.at[pl.ds(brow, 32)], si_b)
            pltpu.sync_copy(dst_h.at[pl.ds(brow, 32)], di_b)
            if split:

                @pl.loop(0, 32)
                def _(i):
                    for jj in range(4):
                        sl = pl.ds(jj * 16, 16)
                        si_b[i, sl] = si_b[i, sl] + off

            eb = web + b * 2048

            def gcopy(buf, sem, cc):
                return pltpu.make_async_copy(xl_h.at[si_b.at[cc]], buf, sem)

            def wcopy(buf, sem, cc):
                return pltpu.make_async_copy(
                    we_h.at[pl.ds(eb + cc * 64, 64)], buf, sem
                )

            gcopy(r0, g0, 0).start()
            wcopy(e0, w0, 0).start()
            gcopy(r1, g1, 1).start()
            wcopy(e1, w1, 1).start()

            def slot(rv, ev, gs, ws, tt, nxt):
                gcopy(rv, gs, tt).wait()
                wcopy(ev, ws, tt).wait()

                @pl.loop(0, 64, step=2)
                def _(i):
                    for ii in range(2):
                        for jj in range(8):
                            sl = pl.ds(jj * 16, 16)
                            rv[i + ii, sl] = rv[i + ii, sl] * ev[i + ii, sl]

                pltpu.sync_copy(rv, agg_sh.at[di_b.at[tt]], add=True)

                @pl.when(nxt < 32)
                def _():
                    gcopy(rv, gs, nxt).start()
                    wcopy(ev, ws, nxt).start()

            @pl.loop(0, 32, step=2)
            def _(t):
                slot(r0, e0, g0, w0, t, t + 2)
                slot(r1, e1, g1, w1, t + 1, t + 3)

        plsc.subcore_barrier()
        pltpu.sync_copy(
            agg_sh.at[pl.ds(zbase, _ROWS_PT)],
            out_h.at[pl.ds(c * _NP + zbase, _ROWS_PT)],
        )

    return k(xl, we, src2, dst2)


def _radial(dl2, layers):
    d2 = dl2.reshape(EP, 1)
    vals = np.linspace(0.0, MAX_RADIUS, NUM_BASIS).astype(np.float32)
    step = float(vals[1] - vals[0])
    hs = [w["fc_w2h"].shape[0] for w in layers]

    def body(d2_ref, *refs):
        wrefs = refs[:9]
        orefs = refs[9:]
        el = jnp.sqrt(d2_ref[...] + 1e-9)
        vgrid = (
            lax.broadcasted_iota(jnp.int32, (1, NUM_BASIS), 1).astype(jnp.float32)
            * step
        )
        diff = (el - vgrid) / step
        emb = (jnp.exp(-diff * diff) / 1.12) * (NUM_BASIS ** 0.5)
        u = 2.0 * (el / MAX_RADIUS - 1.0)
        y = (1.0 - jnp.cos(np.float32(math.pi) * u)) / 2.0
        y = jnp.where(u > 0.0, 0.0, y)
        y = jnp.where(u < -1.0, 1.0, y)
        for li in range(3):
            f0, f1, f2 = wrefs[3 * li : 3 * li + 3]
            h0 = _silu(jnp.dot(emb, f0[...], preferred_element_type=jnp.float32) / np.sqrt(NUM_BASIS))
            h1 = _silu(jnp.dot(h0, f1[...], preferred_element_type=jnp.float32) / np.sqrt(128.0))
            for hh in range(hs[li]):
                w = jnp.dot(h1, f2[hh], preferred_element_type=jnp.float32) / np.sqrt(128.0)
                if hs[li] == 1:
                    orefs[li][...] = w * y
                else:
                    orefs[li][hh] = w * y

    in_specs = [pl.BlockSpec((_BE, 1), lambda i: (i, 0))]
    args = [d2]
    for w in layers:
        args += [w["fc_w0"], w["fc_w1"], w["fc_w2h"]]
        in_specs += [
            pl.BlockSpec((NUM_BASIS, 128), lambda i: (0, 0)),
            pl.BlockSpec((128, 128), lambda i: (0, 0)),
            pl.BlockSpec((w["fc_w2h"].shape[0], 128, 128), lambda i: (0, 0, 0)),
        ]
    out_shapes = []
    out_specs = []
    for h in hs:
        if h == 1:
            out_shapes.append(jax.ShapeDtypeStruct((EP, 128), jnp.float32))
            out_specs.append(pl.BlockSpec((_BE, 128), lambda i: (i, 0)))
        else:
            out_shapes.append(jax.ShapeDtypeStruct((h, EP, 128), jnp.float32))
            out_specs.append(pl.BlockSpec((h, _BE, 128), lambda i: (0, i, 0)))
    outs = pl.pallas_call(
        body,
        grid=(EP // _BE,),
        in_specs=in_specs,
        out_specs=out_specs,
        out_shape=out_shapes,
    )(*args)
    return [o.reshape(-1, 128) for o in outs]


def _pre(h, wsc, w1h, h2=None, p=None):
    d_in = h.shape[1]
    d_out = wsc.shape[1]
    H = w1h.shape[0]

    def body(*refs):
        if p is None:
            h_ref, wsc_ref, w1_ref, xs_ref, xl_ref = refs
            hb = h_ref[...]
        else:
            h_ref, h2_ref, p_ref, wsc_ref, w1_ref, xs_ref, xl_ref = refs
            pv = p_ref[0, 0]
            hb = pv * h_ref[...] + (1.0 - pv) * h2_ref[...]
        c_s = np.float32(math.sin(math.pi / 8))
        xs_ref[...] = c_s * (
            jnp.dot(hb, wsc_ref[...], preferred_element_type=jnp.float32) / np.sqrt(d_in)
        )
        for k in range(H):
            xl_ref[k] = (
                jnp.dot(hb, w1_ref[k], preferred_element_type=jnp.float32) / np.sqrt(d_in)
            )

    in_specs = [pl.BlockSpec((_BN, d_in), lambda i: (i, 0))]
    args = [h]
    if p is not None:
        in_specs += [
            pl.BlockSpec((_BN, d_in), lambda i: (i, 0)),
            pl.BlockSpec((1, 1), lambda i: (0, 0)),
        ]
        args += [h2, p.reshape(1, 1)]
    in_specs += [
        pl.BlockSpec((d_in, d_out), lambda i: (0, 0)),
        pl.BlockSpec((H, d_in, 128), lambda i: (0, 0, 0)),
    ]
    args += [wsc, w1h]
    xs, xl = pl.pallas_call(
        body,
        grid=(N // _BN,),
        in_specs=in_specs,
        out_specs=[
            pl.BlockSpec((_BN, d_out), lambda i: (i, 0)),
            pl.BlockSpec((H, _BN, 128), lambda i: (0, i, 0)),
        ],
        out_shape=[
            jax.ShapeDtypeStruct((N, d_out), jnp.float32),
            jax.ShapeDtypeStruct((H, N, 128), jnp.float32),
        ],
    )(*args)
    return xs, xl.reshape(H * N, 128)


def _combine(agg, xs, w2h, d_in, act):
    d_out = xs.shape[1]
    H = w2h.shape[0]
    a = agg.reshape(2, _NP, 128)

    def body(a_ref, xs_ref, w2_ref, o_ref):
        c_x = np.float32(math.cos(math.pi / 8))
        if H == 1:
            ag = (a_ref[0] + a_ref[1]) / np.sqrt(32.0)
            out = jnp.dot(ag, w2_ref[0], preferred_element_type=jnp.float32)
        else:
            out = jnp.dot(
                a_ref[0] / np.sqrt(32.0), w2_ref[0], preferred_element_type=jnp.float32
            ) + jnp.dot(
                a_ref[1] / np.sqrt(32.0), w2_ref[1], preferred_element_type=jnp.float32
            )
        o = xs_ref[...] + c_x * (out / np.sqrt(d_in))
        if act:
            o = _silu(o)
        o_ref[...] = o

    return pl.pallas_call(
        body,
        grid=(N // _BN,),
        in_specs=[
            pl.BlockSpec((2, _BN, 128), lambda i: (0, i, 0)),
            pl.BlockSpec((_BN, d_out), lambda i: (i, 0)),
            pl.BlockSpec((H, 128, d_out), lambda i: (0, 0, 0)),
        ],
        out_specs=pl.BlockSpec((_BN, d_out), lambda i: (i, 0)),
        out_shape=jax.ShapeDtypeStruct((N, d_out), jnp.float32),
    )(a, xs, w2h)


def _prep_params(layers):
    out = []
    for lp in layers:
        d_in = lp["W_sc"].shape[0]
        H = d_in // 128
        w1 = lp["W_lin1"].reshape(d_in, H, 128)
        f2 = lp["fc_w2"].reshape(128, H, 128)
        out.append(
            {
                "W_sc": lp["W_sc"],
                "W_lin1h": jnp.transpose(w1, (1, 0, 2)),
                "fc_w0": lp["fc_w0"],
                "fc_w1": lp["fc_w1"],
                "fc_w2h": jnp.transpose(f2, (1, 0, 2)),
                "W_lin2h": lp["W_lin2"].reshape(H, 128, -1),
                "H": H,
                "d_in": d_in,
            }
        )
    return out


def _network(h, pos, srcp, src2, dst2, layers, h2=None, p=None):
    dl2 = _geom(pos[:, 0], pos[:, 1], pos[:, 2], srcp[0], srcp[1])
    wes = _radial(dl2, layers)
    for li, lp in enumerate(layers):
        if li == 0 and p is not None:
            xs, xl = _pre(h, lp["W_sc"], lp["W_lin1h"], h2=h2, p=p)
        else:
            xs, xl = _pre(h, lp["W_sc"], lp["W_lin1h"])
        agg = _sparse_layer(lp["H"] == 2, xl, wes[li], src2, dst2)
        h = _combine(agg, xs, lp["W_lin2h"], lp["d_in"], act=(li < 2))
    return h


def kernel(x, x_final_state, pos, pos_final_state, pos_interpolated_transition_state, p, edge_index, batch, params):
    src = edge_index[0].astype(jnp.int32)
    dst = edge_index[1].astype(jnp.int32)
    srcp = jnp.concatenate([src, jnp.zeros((EP - E,), jnp.int32)])
    dstp = jnp.concatenate([dst, jnp.full((EP - E,), N, jnp.int32)])
    src2, dst2 = srcp.reshape(EP // 64, 64), dstp.reshape(EP // 64, 64)
    sp = (srcp, dstp)

    net_i = _prep_params(params["net_init"])
    net_f = _prep_params(params["net_final"])
    net_ts = _prep_params(params["net_ts"])
    out_i = _network(x, pos, sp, src2, dst2, net_i)
    out_f = _network(x_final_state, pos_final_state, sp, src2, dst2, net_f)
    out_ts = _network(
        out_i,
        pos_interpolated_transition_state,
        sp,
        src2,
        dst2,
        net_ts,
        h2=out_f,
        p=p[0],
    )
    return out_ts


# speedup vs baseline: 1.1755x; 1.0006x over previous
import dataclasses
import functools
import math

import jax
import jax.numpy as jnp
import numpy as np
from jax import lax
from jax.experimental import pallas as pl
from jax.experimental.pallas import tpu as pltpu
from jax.experimental.pallas import tpu_sc as plsc

N = 10000
E = 320000
EP = 327680
NUM_BASIS = 10
MAX_RADIUS = 5.0

_MESH = plsc.VectorSubcoreMesh(core_axis_name="c", subcore_axis_name="s")
_SC_PARAMS = pltpu.CompilerParams()
if "needs_layout_passes" in pltpu.CompilerParams.__dataclass_fields__:
    _SC_PARAMS = dataclasses.replace(_SC_PARAMS, needs_layout_passes=False)
_NS = 16
_EPT_G = E // 32
_ROWS_PT = 632
_NP = _ROWS_PT * _NS

_BE = 2048
_BN = 2000


def _silu(v):
    return v * jax.nn.sigmoid(v)


def _geom(px, py, pz, src, dst):
    @functools.partial(
        pl.kernel,
        out_type=jax.ShapeDtypeStruct((EP,), jnp.float32),
        mesh=_MESH,
        compiler_params=_SC_PARAMS,
        scratch_types=[
            pltpu.VMEM((N,), jnp.float32),
            pltpu.VMEM((N,), jnp.float32),
            pltpu.VMEM((N,), jnp.float32),
            pltpu.VMEM((_EPT_G,), jnp.int32),
            pltpu.VMEM((_EPT_G,), jnp.int32),
            pltpu.VMEM((_EPT_G,), jnp.float32),
        ],
    )
    def k(px_h, py_h, pz_h, src_h, dst_h, out_h, px_v, py_v, pz_v, si_v, di_v, o_v):
        c = lax.axis_index("c")
        s = lax.axis_index("s")
        base = (c * _NS + s) * _EPT_G
        pltpu.sync_copy(px_h, px_v)
        pltpu.sync_copy(py_h, py_v)
        pltpu.sync_copy(pz_h, pz_v)
        pltpu.sync_copy(src_h.at[pl.ds(base, _EPT_G)], si_v)
        pltpu.sync_copy(dst_h.at[pl.ds(base, _EPT_G)], di_v)

        def _do16(i):
            sl = pl.ds(i, 16)
            a = si_v[sl]
            b = di_v[sl]
            dx = plsc.load_gather(px_v, [a]) - plsc.load_gather(px_v, [b])
            dy = plsc.load_gather(py_v, [a]) - plsc.load_gather(py_v, [b])
            dz = plsc.load_gather(pz_v, [a]) - plsc.load_gather(pz_v, [b])
            o_v[sl] = dx * dx + dy * dy + dz * dz

        @pl.loop(0, _EPT_G - 16, step=32)
        def _(i):
            _do16(i)
            _do16(i + 16)

        _do16(_EPT_G - 16)
        pltpu.sync_copy(o_v, out_h.at[pl.ds(base, _EPT_G)])

    return k(px, py, pz, src, dst)


_NR = EP // 128


def _sparse_layer(split, xl, we, src2, dst2):
    nchunks = (_NR * 2) // _NS if split else _NR // _NS
    nblocks = nchunks // 32

    @functools.partial(
        pl.kernel,
        out_type=jax.ShapeDtypeStruct((2 * _NP, 128), jnp.float32),
        mesh=_MESH,
        compiler_params=_SC_PARAMS,
        scratch_types=[
            pltpu.VMEM((32, 64), jnp.int32),
            pltpu.VMEM((32, 64), jnp.int32),
            pltpu.VMEM((64, 128), jnp.float32),
            pltpu.VMEM((64, 128), jnp.float32),
            pltpu.VMEM((64, 128), jnp.float32),
            pltpu.VMEM((64, 128), jnp.float32),
            pltpu.SemaphoreType.DMA,
            pltpu.SemaphoreType.DMA,
            pltpu.SemaphoreType.DMA,
            pltpu.SemaphoreType.DMA,
            pltpu.VMEM_SHARED((_NP, 128), jnp.float32),
        ],
    )
    def k(xl_h, we_h, src_h, dst_h, out_h, si_b, di_b, r0, r1, e0, e1,
          g0, g1, w0, w1, agg_sh):
        c = lax.axis_index("c")
        s = lax.axis_index("s")
        cbase = (s * nchunks) if split else ((c * _NS + s) * nchunks)
        ebase = cbase * 64

        @pl.loop(0, 64)
        def _(i):
            for jj in range(8):
                r0[i, pl.ds(jj * 16, 16)] = jnp.zeros((16,), jnp.float32)

        zbase = s * _ROWS_PT
        for t in range(9):
            pltpu.sync_copy(r0, agg_sh.at[pl.ds(zbase + t * 64, 64)])
        pltpu.sync_copy(r0.at[pl.ds(0, 56)], agg_sh.at[pl.ds(zbase + 576, 56)])
        plsc.subcore_barrier()

        web = (c * EP if split else 0) + ebase
        off = c * N

        @pl.loop(0, nblocks)
        def _(b):
            brow = cbase + b * 32
            pltpu.sync_copy(src_h.at[pl.ds(brow, 32)], si_b)
            pltpu.sync_copy(dst_h.at[pl.ds(brow, 32)], di_b)
            if split:

                @pl.loop(0, 32)
                def _(i):
                    for jj in range(4):
                        sl = pl.ds(jj * 16, 16)
                        si_b[i, sl] = si_b[i, sl] + off

            eb = web + b * 2048

            def gcopy(buf, sem, cc):
                return pltpu.make_async_copy(xl_h.at[si_b.at[cc]], buf, sem)

            def wcopy(buf, sem, cc):
                return pltpu.make_async_copy(
                    we_h.at[pl.ds(eb + cc * 64, 64)], buf, sem
                )

            gcopy(r0, g0, 0).start()
            wcopy(e0, w0, 0).start()
            gcopy(r1, g1, 1).start()
            wcopy(e1, w1, 1).start()

            def slot(rv, ev, gs, ws, tt, nxt):
                gcopy(rv, gs, tt).wait()
                wcopy(ev, ws, tt).wait()

                @pl.loop(0, 64, step=4)
                def _(i):
                    for ii in range(4):
                        for jj in range(8):
                            sl = pl.ds(jj * 16, 16)
                            rv[i + ii, sl] = rv[i + ii, sl] * ev[i + ii, sl]

                pltpu.sync_copy(rv, agg_sh.at[di_b.at[tt]], add=True)

                @pl.when(nxt < 32)
                def _():
                    gcopy(rv, gs, nxt).start()
                    wcopy(ev, ws, nxt).start()

            @pl.loop(0, 32, step=2)
            def _(t):
                slot(r0, e0, g0, w0, t, t + 2)
                slot(r1, e1, g1, w1, t + 1, t + 3)

        plsc.subcore_barrier()
        pltpu.sync_copy(
            agg_sh.at[pl.ds(zbase, _ROWS_PT)],
            out_h.at[pl.ds(c * _NP + zbase, _ROWS_PT)],
        )

    return k(xl, we, src2, dst2)


def _radial(dl2, layers):
    d2 = dl2.reshape(EP, 1)
    vals = np.linspace(0.0, MAX_RADIUS, NUM_BASIS).astype(np.float32)
    step = float(vals[1] - vals[0])
    hs = [w["fc_w2h"].shape[0] for w in layers]

    def body(d2_ref, *refs):
        wrefs = refs[:9]
        orefs = refs[9:]
        el = jnp.sqrt(d2_ref[...] + 1e-9)
        vgrid = (
            lax.broadcasted_iota(jnp.int32, (1, NUM_BASIS), 1).astype(jnp.float32)
            * step
        )
        diff = (el - vgrid) / step
        emb = (jnp.exp(-diff * diff) / 1.12) * (NUM_BASIS ** 0.5)
        u = 2.0 * (el / MAX_RADIUS - 1.0)
        y = (1.0 - jnp.cos(np.float32(math.pi) * u)) / 2.0
        y = jnp.where(u > 0.0, 0.0, y)
        y = jnp.where(u < -1.0, 1.0, y)
        for li in range(3):
            f0, f1, f2 = wrefs[3 * li : 3 * li + 3]
            h0 = _silu(jnp.dot(emb, f0[...], preferred_element_type=jnp.float32) / np.sqrt(NUM_BASIS))
            h1 = _silu(jnp.dot(h0, f1[...], preferred_element_type=jnp.float32) / np.sqrt(128.0))
            for hh in range(hs[li]):
                w = jnp.dot(h1, f2[hh], preferred_element_type=jnp.float32) / np.sqrt(128.0)
                if hs[li] == 1:
                    orefs[li][...] = w * y
                else:
                    orefs[li][hh] = w * y

    in_specs = [pl.BlockSpec((_BE, 1), lambda i: (i, 0))]
    args = [d2]
    for w in layers:
        args += [w["fc_w0"], w["fc_w1"], w["fc_w2h"]]
        in_specs += [
            pl.BlockSpec((NUM_BASIS, 128), lambda i: (0, 0)),
            pl.BlockSpec((128, 128), lambda i: (0, 0)),
            pl.BlockSpec((w["fc_w2h"].shape[0], 128, 128), lambda i: (0, 0, 0)),
        ]
    out_shapes = []
    out_specs = []
    for h in hs:
        if h == 1:
            out_shapes.append(jax.ShapeDtypeStruct((EP, 128), jnp.float32))
            out_specs.append(pl.BlockSpec((_BE, 128), lambda i: (i, 0)))
        else:
            out_shapes.append(jax.ShapeDtypeStruct((h, EP, 128), jnp.float32))
            out_specs.append(pl.BlockSpec((h, _BE, 128), lambda i: (0, i, 0)))
    outs = pl.pallas_call(
        body,
        grid=(EP // _BE,),
        in_specs=in_specs,
        out_specs=out_specs,
        out_shape=out_shapes,
    )(*args)
    return [o.reshape(-1, 128) for o in outs]


def _pre(h, wsc, w1h, h2=None, p=None):
    d_in = h.shape[1]
    d_out = wsc.shape[1]
    H = w1h.shape[0]

    def body(*refs):
        if p is None:
            h_ref, wsc_ref, w1_ref, xs_ref, xl_ref = refs
            hb = h_ref[...]
        else:
            h_ref, h2_ref, p_ref, wsc_ref, w1_ref, xs_ref, xl_ref = refs
            pv = p_ref[0, 0]
            hb = pv * h_ref[...] + (1.0 - pv) * h2_ref[...]
        c_s = np.float32(math.sin(math.pi / 8))
        xs_ref[...] = c_s * (
            jnp.dot(hb, wsc_ref[...], preferred_element_type=jnp.float32) / np.sqrt(d_in)
        )
        for k in range(H):
            xl_ref[k] = (
                jnp.dot(hb, w1_ref[k], preferred_element_type=jnp.float32) / np.sqrt(d_in)
            )

    in_specs = [pl.BlockSpec((_BN, d_in), lambda i: (i, 0))]
    args = [h]
    if p is not None:
        in_specs += [
            pl.BlockSpec((_BN, d_in), lambda i: (i, 0)),
            pl.BlockSpec((1, 1), lambda i: (0, 0)),
        ]
        args += [h2, p.reshape(1, 1)]
    in_specs += [
        pl.BlockSpec((d_in, d_out), lambda i: (0, 0)),
        pl.BlockSpec((H, d_in, 128), lambda i: (0, 0, 0)),
    ]
    args += [wsc, w1h]
    xs, xl = pl.pallas_call(
        body,
        grid=(N // _BN,),
        in_specs=in_specs,
        out_specs=[
            pl.BlockSpec((_BN, d_out), lambda i: (i, 0)),
            pl.BlockSpec((H, _BN, 128), lambda i: (0, i, 0)),
        ],
        out_shape=[
            jax.ShapeDtypeStruct((N, d_out), jnp.float32),
            jax.ShapeDtypeStruct((H, N, 128), jnp.float32),
        ],
    )(*args)
    return xs, xl.reshape(H * N, 128)


def _combine(agg, xs, w2h, d_in, act):
    d_out = xs.shape[1]
    H = w2h.shape[0]
    a = agg.reshape(2, _NP, 128)

    def body(a_ref, xs_ref, w2_ref, o_ref):
        c_x = np.float32(math.cos(math.pi / 8))
        if H == 1:
            ag = (a_ref[0] + a_ref[1]) / np.sqrt(32.0)
            out = jnp.dot(ag, w2_ref[0], preferred_element_type=jnp.float32)
        else:
            out = jnp.dot(
                a_ref[0] / np.sqrt(32.0), w2_ref[0], preferred_element_type=jnp.float32
            ) + jnp.dot(
                a_ref[1] / np.sqrt(32.0), w2_ref[1], preferred_element_type=jnp.float32
            )
        o = xs_ref[...] + c_x * (out / np.sqrt(d_in))
        if act:
            o = _silu(o)
        o_ref[...] = o

    return pl.pallas_call(
        body,
        grid=(N // _BN,),
        in_specs=[
            pl.BlockSpec((2, _BN, 128), lambda i: (0, i, 0)),
            pl.BlockSpec((_BN, d_out), lambda i: (i, 0)),
            pl.BlockSpec((H, 128, d_out), lambda i: (0, 0, 0)),
        ],
        out_specs=pl.BlockSpec((_BN, d_out), lambda i: (i, 0)),
        out_shape=jax.ShapeDtypeStruct((N, d_out), jnp.float32),
    )(a, xs, w2h)


def _prep_params(layers):
    out = []
    for lp in layers:
        d_in = lp["W_sc"].shape[0]
        H = d_in // 128
        w1 = lp["W_lin1"].reshape(d_in, H, 128)
        f2 = lp["fc_w2"].reshape(128, H, 128)
        out.append(
            {
                "W_sc": lp["W_sc"],
                "W_lin1h": jnp.transpose(w1, (1, 0, 2)),
                "fc_w0": lp["fc_w0"],
                "fc_w1": lp["fc_w1"],
                "fc_w2h": jnp.transpose(f2, (1, 0, 2)),
                "W_lin2h": lp["W_lin2"].reshape(H, 128, -1),
                "H": H,
                "d_in": d_in,
            }
        )
    return out


def _network(h, pos, srcp, src2, dst2, layers, h2=None, p=None):
    dl2 = _geom(pos[:, 0], pos[:, 1], pos[:, 2], srcp[0], srcp[1])
    wes = _radial(dl2, layers)
    for li, lp in enumerate(layers):
        if li == 0 and p is not None:
            xs, xl = _pre(h, lp["W_sc"], lp["W_lin1h"], h2=h2, p=p)
        else:
            xs, xl = _pre(h, lp["W_sc"], lp["W_lin1h"])
        agg = _sparse_layer(lp["H"] == 2, xl, wes[li], src2, dst2)
        h = _combine(agg, xs, lp["W_lin2h"], lp["d_in"], act=(li < 2))
    return h


def kernel(x, x_final_state, pos, pos_final_state, pos_interpolated_transition_state, p, edge_index, batch, params):
    src = edge_index[0].astype(jnp.int32)
    dst = edge_index[1].astype(jnp.int32)
    srcp = jnp.concatenate([src, jnp.zeros((EP - E,), jnp.int32)])
    dstp = jnp.concatenate([dst, jnp.full((EP - E,), N, jnp.int32)])
    src2, dst2 = srcp.reshape(EP // 64, 64), dstp.reshape(EP // 64, 64)
    sp = (srcp, dstp)

    net_i = _prep_params(params["net_init"])
    net_f = _prep_params(params["net_final"])
    net_ts = _prep_params(params["net_ts"])
    out_i = _network(x, pos, sp, src2, dst2, net_i)
    out_f = _network(x_final_state, pos_final_state, sp, src2, dst2, net_f)
    out_ts = _network(
        out_i,
        pos_interpolated_transition_state,
        sp,
        src2,
        dst2,
        net_ts,
        h2=out_f,
        p=p[0],
    )
    return out_ts
